# baseline (speedup 1.0000x reference)
import functools

import jax
import jax.numpy as jnp
from jax import lax
from jax.experimental import pallas as pl
from jax.experimental.pallas import tpu as pltpu
from jax.experimental.pallas import tpu_sc as plsc

N_ROWS = 16384
N_COLS = 1000
ALIGNC = 896
L = 16
NC = 2
NS = 16
NW = NC * NS
ROWS_PER_W = N_ROWS // NW
CH = 32
N_CH = ROWS_PER_W // CH


def _sc_body(probs_hbm, tgt_hbm, rwd_hbm, out_hbm,
             tgt_v, rwd_v, buf0, buf1, acc_v, red_v, shared, sem0, sem1):
    c = lax.axis_index("c")
    s = lax.axis_index("s")
    w = c * NS + s
    base = w * ROWS_PER_W

    pltpu.sync_copy(tgt_hbm.at[pl.ds(base, ROWS_PER_W)], tgt_v)
    pltpu.sync_copy(rwd_hbm.at[pl.ds(base, ROWS_PER_W)], rwd_v)

    lane = lax.iota(jnp.int32, L)
    bufs = (buf0, buf1)
    sems = (sem0, sem1)

    def start(chunk, buf, sem):
        row0 = pl.multiple_of(base + chunk * CH, 8)
        pltpu.async_copy(
            probs_hbm.at[pl.ds(row0, CH), pl.ds(0, ALIGNC)], buf, sem)

    def drain(buf, sem):
        pltpu.make_async_copy(
            probs_hbm.at[pl.ds(0, CH), pl.ds(0, ALIGNC)], buf, sem).wait()

    start(0, buf0, sem0)
    start(1, buf1, sem1)

    def pair(m, acc):
        for b in range(2):
            chunk = 2 * m + b
            drain(bufs[b], sems[b])
            start(jnp.minimum(chunk + 2, N_CH - 2 + b), bufs[b], sems[b])
            for half in range(CH // L):
                off = pl.multiple_of(chunk * CH + half * L, 8)
                tv = tgt_v[pl.ds(off, L)]
                rv = rwd_v[pl.ds(off, L)]
                for i in range(L):
                    t_k = tv[i]
                    group = pl.multiple_of(
                        jnp.minimum(t_k & ~15, ALIGNC - L), 16)
                    v16 = bufs[b][half * L + i, pl.ds(group, L)]
                    rk = jnp.where(t_k < ALIGNC, rv[i], 0.0)
                    acc = acc + jnp.where(lane == (t_k & 15),
                                          (1.0 - v16) * rk, 0.0)
        return acc

    acc = lax.fori_loop(0, N_CH // 2, pair, jnp.zeros((L,), jnp.float32))
    drain(buf0, sem0)
    drain(buf1, sem1)
    acc_v[...] = acc * (1.0 / N_ROWS)

    pltpu.sync_copy(acc_v, shared.at[pl.ds(s * L, L)])
    plsc.subcore_barrier()

    @pl.when(s == 0)
    def _reduce():
        pltpu.sync_copy(shared, red_v)
        tot = jnp.zeros((L,), jnp.float32)
        for k in range(NS):
            tot = tot + red_v[pl.ds(k * L, L)]
        acc_v[...] = tot
        pltpu.sync_copy(acc_v, out_hbm.at[pl.ds(c * L, L)])


_ganloss_sc = functools.partial(
    pl.kernel,
    out_type=jax.ShapeDtypeStruct((NC * L,), jnp.float32),
    mesh=plsc.VectorSubcoreMesh(core_axis_name="c", subcore_axis_name="s"),
    scratch_types=[
        pltpu.VMEM((ROWS_PER_W,), jnp.int32),
        pltpu.VMEM((ROWS_PER_W,), jnp.float32),
        pltpu.VMEM((CH, ALIGNC), jnp.float32),
        pltpu.VMEM((CH, ALIGNC), jnp.float32),
        pltpu.VMEM((L,), jnp.float32),
        pltpu.VMEM((NS * L,), jnp.float32),
        pltpu.VMEM_SHARED((NS * L,), jnp.float32),
        pltpu.SemaphoreType.DMA,
        pltpu.SemaphoreType.DMA,
    ],
)(_sc_body)


BLK = 1024
GRID = N_ROWS // BLK
TCW = 128
TC_COL0 = 7 * TCW


def _tc_tail_body(tgt_ref, rwd_ref, probs_ref, out_ref):
    g = pl.program_id(0)
    p = probs_ref[...]
    t = tgt_ref[...]
    r = rwd_ref[...]
    tailf = jnp.where(t >= TC_COL0, 1.0, 0.0)
    rt = r * tailf
    cols = jax.lax.broadcasted_iota(jnp.int32, (BLK, TCW), 1) + TC_COL0
    psel = jnp.where(cols == t, p * rt, 0.0)
    part = (jnp.sum(rt) - jnp.sum(psel)) * (1.0 / N_ROWS)

    @pl.when(g == 0)
    def _init():
        out_ref[0, 0] = 0.0

    out_ref[0, 0] += part


_ganloss_tc_tail = pl.pallas_call(
    _tc_tail_body,
    grid=(GRID,),
    in_specs=[
        pl.BlockSpec((BLK, 1), lambda g: (g, 0)),
        pl.BlockSpec((BLK, 1), lambda g: (g, 0)),
        pl.BlockSpec((BLK, TCW), lambda g: (g, 7)),
    ],
    out_specs=pl.BlockSpec((1, 1), lambda g: (0, 0), memory_space=pltpu.SMEM),
    out_shape=jax.ShapeDtypeStruct((1, 1), jnp.float32),
    compiler_params=pltpu.CompilerParams(
        dimension_semantics=("arbitrary",),
    ),
)


def kernel(probs, targets, reward):
    t32 = targets.astype(jnp.int32)
    sc_partials = _ganloss_sc(probs, t32, reward)
    tc_part = _ganloss_tc_tail(
        t32.reshape(N_ROWS, 1), reward.reshape(N_ROWS, 1), probs)
    return jnp.sum(sc_partials) + tc_part[0, 0]


# speedup vs baseline: 1.0126x; 1.0126x over previous
import functools

import jax
import jax.numpy as jnp
from jax import lax
from jax.experimental import pallas as pl
from jax.experimental.pallas import tpu as pltpu
from jax.experimental.pallas import tpu_sc as plsc

N_ROWS = 16384
N_COLS = 1000
ALIGNC = 896
L = 16
NC = 2
NS = 16
NW = NC * NS
ROWS_PER_W = N_ROWS // NW
CH = 16
N_CH = ROWS_PER_W // CH


def _sc_body(probs_hbm, tgt_hbm, rwd_hbm, out_hbm,
             tgt_v, rwd_v, buf0, buf1, acc_v, red_v, shared, sem0, sem1):
    c = lax.axis_index("c")
    s = lax.axis_index("s")
    w = c * NS + s
    base = w * ROWS_PER_W

    pltpu.sync_copy(tgt_hbm.at[pl.ds(base, ROWS_PER_W)], tgt_v)
    pltpu.sync_copy(rwd_hbm.at[pl.ds(base, ROWS_PER_W)], rwd_v)

    lane = lax.iota(jnp.int32, L)
    bufs = (buf0, buf1)
    sems = (sem0, sem1)

    def start(chunk, buf, sem):
        row0 = pl.multiple_of(base + chunk * CH, 8)
        pltpu.async_copy(
            probs_hbm.at[pl.ds(row0, CH), pl.ds(0, ALIGNC)], buf, sem)

    def drain(buf, sem):
        pltpu.make_async_copy(
            probs_hbm.at[pl.ds(0, CH), pl.ds(0, ALIGNC)], buf, sem).wait()

    start(0, buf0, sem0)
    start(1, buf1, sem1)

    def pair(m, acc):
        for b in range(2):
            chunk = 2 * m + b
            drain(bufs[b], sems[b])
            start(jnp.minimum(chunk + 2, N_CH - 2 + b), bufs[b], sems[b])
            off = pl.multiple_of(chunk * CH, 8)
            tv = tgt_v[pl.ds(off, L)]
            rv = rwd_v[pl.ds(off, L)]
            for i in range(CH):
                t_k = tv[i]
                group = pl.multiple_of(jnp.minimum(t_k & ~15, ALIGNC - L), 16)
                v16 = bufs[b][i, pl.ds(group, L)]
                rk = jnp.where(t_k < ALIGNC, rv[i], 0.0)
                acc = acc + jnp.where(lane == (t_k & 15),
                                      (1.0 - v16) * rk, 0.0)
        return acc

    acc = lax.fori_loop(0, N_CH // 2, pair, jnp.zeros((L,), jnp.float32))
    drain(buf0, sem0)
    drain(buf1, sem1)
    acc_v[...] = acc * (1.0 / N_ROWS)

    pltpu.sync_copy(acc_v, shared.at[pl.ds(s * L, L)])
    plsc.subcore_barrier()

    @pl.when(s == 0)
    def _reduce():
        pltpu.sync_copy(shared, red_v)
        tot = jnp.zeros((L,), jnp.float32)
        for k in range(NS):
            tot = tot + red_v[pl.ds(k * L, L)]
        acc_v[...] = tot
        pltpu.sync_copy(acc_v, out_hbm.at[pl.ds(c * L, L)])


_ganloss_sc = functools.partial(
    pl.kernel,
    out_type=jax.ShapeDtypeStruct((NC * L,), jnp.float32),
    mesh=plsc.VectorSubcoreMesh(core_axis_name="c", subcore_axis_name="s"),
    scratch_types=[
        pltpu.VMEM((ROWS_PER_W,), jnp.int32),
        pltpu.VMEM((ROWS_PER_W,), jnp.float32),
        pltpu.VMEM((CH, ALIGNC), jnp.float32),
        pltpu.VMEM((CH, ALIGNC), jnp.float32),
        pltpu.VMEM((L,), jnp.float32),
        pltpu.VMEM((NS * L,), jnp.float32),
        pltpu.VMEM_SHARED((NS * L,), jnp.float32),
        pltpu.SemaphoreType.DMA,
        pltpu.SemaphoreType.DMA,
    ],
)(_sc_body)


BLK = 1024
GRID = N_ROWS // BLK
TCW = 128
TC_COL0 = 7 * TCW


def _tc_tail_body(tgt_ref, rwd_ref, probs_ref, out_ref):
    g = pl.program_id(0)
    p = probs_ref[...]
    t = tgt_ref[...]
    r = rwd_ref[...]
    tailf = jnp.where(t >= TC_COL0, 1.0, 0.0)
    rt = r * tailf
    cols = jax.lax.broadcasted_iota(jnp.int32, (BLK, TCW), 1) + TC_COL0
    psel = jnp.where(cols == t, p * rt, 0.0)
    part = (jnp.sum(rt) - jnp.sum(psel)) * (1.0 / N_ROWS)

    @pl.when(g == 0)
    def _init():
        out_ref[0, 0] = 0.0

    out_ref[0, 0] += part


_ganloss_tc_tail = pl.pallas_call(
    _tc_tail_body,
    grid=(GRID,),
    in_specs=[
        pl.BlockSpec((BLK, 1), lambda g: (g, 0)),
        pl.BlockSpec((BLK, 1), lambda g: (g, 0)),
        pl.BlockSpec((BLK, TCW), lambda g: (g, 7)),
    ],
    out_specs=pl.BlockSpec((1, 1), lambda g: (0, 0), memory_space=pltpu.SMEM),
    out_shape=jax.ShapeDtypeStruct((1, 1), jnp.float32),
    compiler_params=pltpu.CompilerParams(
        dimension_semantics=("arbitrary",),
    ),
)


def kernel(probs, targets, reward):
    t32 = targets.astype(jnp.int32)
    sc_partials = _ganloss_sc(probs, t32, reward)
    tc_part = _ganloss_tc_tail(
        t32.reshape(N_ROWS, 1), reward.reshape(N_ROWS, 1), probs)
    return jnp.sum(sc_partials) + tc_part[0, 0]
